# baseline (speedup 1.0000x reference)
import functools
import math

import jax
import jax.numpy as jnp
from jax import lax
from jax.experimental import pallas as pl
from jax.experimental.pallas import tpu as pltpu
from jax.experimental.pallas import tpu_sc as plsc

EMBED_DIM = 128
SCALE = math.sqrt(EMBED_DIM)

NUM_CORES = 2
NUM_SUBCORES = 16
NUM_WORKERS = NUM_CORES * NUM_SUBCORES

CHUNK = 128
NBUF = 6
GAHEAD = 3


def _scale_body(t_ref, o_ref):
    o_ref[...] = t_ref[...] * SCALE


def _prescale(table):
    vocab, d = table.shape
    block = 25000
    return pl.pallas_call(
        _scale_body,
        out_shape=jax.ShapeDtypeStruct((vocab, d), table.dtype),
        grid=(vocab // block,),
        in_specs=[pl.BlockSpec((block, d), lambda i: (i, 0))],
        out_specs=pl.BlockSpec((block, d), lambda i: (i, 0)),
    )(table)


def _gather_fn(b_per_w, n_chunks, idx_hbm, table_hbm, out_hbm,
               idx_v, rows_v, gsem, ssem):
    wid = lax.axis_index("s") * NUM_CORES + lax.axis_index("c")
    base = wid * b_per_w

    pltpu.sync_copy(idx_hbm.at[pl.ds(base, b_per_w)], idx_v)

    def idx_slice(i):
        return idx_v.at[pl.ds(i * CHUNK, CHUNK)]

    def start_gather(i, buf):
        pltpu.async_copy(table_hbm.at[idx_slice(i)], rows_v.at[buf],
                         gsem.at[buf])

    def wait_gather(buf):
        pltpu.make_async_copy(table_hbm.at[idx_slice(0)], rows_v.at[buf],
                              gsem.at[buf]).wait()

    def start_scatter(i, buf):
        pltpu.async_copy(rows_v.at[buf], out_hbm.at[pl.ds(base + i * CHUNK, CHUNK)],
                         ssem.at[buf])

    def wait_scatter(buf):
        pltpu.make_async_copy(rows_v.at[buf], out_hbm.at[pl.ds(0, CHUNK)],
                              ssem.at[buf]).wait()

    for g in range(min(GAHEAD, n_chunks)):
        start_gather(g, g)

    def body(i, carry):
        buf = lax.rem(i, NBUF)
        gbuf = lax.rem(i + GAHEAD, NBUF)

        @pl.when(i + GAHEAD < n_chunks)
        def _():
            @pl.when(i + GAHEAD >= NBUF)
            def _():
                wait_scatter(gbuf)
            start_gather(i + GAHEAD, gbuf)

        wait_gather(buf)
        start_scatter(i, buf)
        return carry

    lax.fori_loop(0, n_chunks, body, 0)
    for k in range(max(0, min(NBUF, n_chunks))):
        wait_scatter((n_chunks - 1 - k) % NBUF)


def _gather(ids_flat, scaled_table):
    n = ids_flat.shape[0]
    b_per_w = n // NUM_WORKERS
    n_chunks = b_per_w // CHUNK
    mesh = plsc.VectorSubcoreMesh(
        core_axis_name="c",
        subcore_axis_name="s",
        num_cores=NUM_CORES,
        num_subcores=NUM_SUBCORES,
    )
    run = pl.kernel(
        functools.partial(_gather_fn, b_per_w, n_chunks),
        out_type=jax.ShapeDtypeStruct((n, EMBED_DIM), jnp.float32),
        mesh=mesh,
        scratch_types=[
            pltpu.VMEM((b_per_w,), jnp.int32),
            pltpu.VMEM((NBUF, CHUNK, EMBED_DIM), jnp.float32),
            pltpu.SemaphoreType.DMA((NBUF,)),
            pltpu.SemaphoreType.DMA((NBUF,)),
        ],
    )
    return run(ids_flat, scaled_table)


def kernel(token_ids, table):
    b, s = token_ids.shape
    ids_flat = token_ids.reshape(-1).astype(jnp.int32)
    out = _gather(ids_flat, _prescale(table))
    return out.reshape(b, s, EMBED_DIM)


# speedup vs baseline: 1.0018x; 1.0018x over previous
import functools
import math

import jax
import jax.numpy as jnp
from jax import lax
from jax.experimental import pallas as pl
from jax.experimental.pallas import tpu as pltpu
from jax.experimental.pallas import tpu_sc as plsc

EMBED_DIM = 128
SCALE = math.sqrt(EMBED_DIM)

NUM_CORES = 2
NUM_SUBCORES = 16
NUM_WORKERS = NUM_CORES * NUM_SUBCORES

CHUNK = 200
NBUF = 4
GAHEAD = 2


def _scale_body(t_ref, o_ref):
    o_ref[...] = t_ref[...] * SCALE


def _prescale(table):
    vocab, d = table.shape
    block = 25000
    return pl.pallas_call(
        _scale_body,
        out_shape=jax.ShapeDtypeStruct((vocab, d), table.dtype),
        grid=(vocab // block,),
        in_specs=[pl.BlockSpec((block, d), lambda i: (i, 0))],
        out_specs=pl.BlockSpec((block, d), lambda i: (i, 0)),
    )(table)


def _gather_fn(b_per_w, n_chunks, idx_hbm, table_hbm, out_hbm,
               idx_v, rows_v, gsem, ssem):
    wid = lax.axis_index("s") * NUM_CORES + lax.axis_index("c")
    base = wid * b_per_w

    pltpu.sync_copy(idx_hbm.at[pl.ds(base, b_per_w)], idx_v)

    def idx_slice(i):
        return idx_v.at[pl.ds(i * CHUNK, CHUNK)]

    def start_gather(i, buf):
        pltpu.async_copy(table_hbm.at[idx_slice(i)], rows_v.at[buf],
                         gsem.at[buf])

    def wait_gather(buf):
        pltpu.make_async_copy(table_hbm.at[idx_slice(0)], rows_v.at[buf],
                              gsem.at[buf]).wait()

    def start_scatter(i, buf):
        pltpu.async_copy(rows_v.at[buf], out_hbm.at[pl.ds(base + i * CHUNK, CHUNK)],
                         ssem.at[buf])

    def wait_scatter(buf):
        pltpu.make_async_copy(rows_v.at[buf], out_hbm.at[pl.ds(0, CHUNK)],
                              ssem.at[buf]).wait()

    for g in range(min(GAHEAD, n_chunks)):
        start_gather(g, g)

    def body(i, carry):
        buf = lax.rem(i, NBUF)
        gbuf = lax.rem(i + GAHEAD, NBUF)

        @pl.when(i + GAHEAD < n_chunks)
        def _():
            @pl.when(i + GAHEAD >= NBUF)
            def _():
                wait_scatter(gbuf)
            start_gather(i + GAHEAD, gbuf)

        wait_gather(buf)
        start_scatter(i, buf)
        return carry

    lax.fori_loop(0, n_chunks, body, 0)
    for k in range(max(0, min(NBUF, n_chunks))):
        wait_scatter((n_chunks - 1 - k) % NBUF)


def _gather(ids_flat, scaled_table):
    n = ids_flat.shape[0]
    b_per_w = n // NUM_WORKERS
    n_chunks = b_per_w // CHUNK
    mesh = plsc.VectorSubcoreMesh(
        core_axis_name="c",
        subcore_axis_name="s",
        num_cores=NUM_CORES,
        num_subcores=NUM_SUBCORES,
    )
    run = pl.kernel(
        functools.partial(_gather_fn, b_per_w, n_chunks),
        out_type=jax.ShapeDtypeStruct((n, EMBED_DIM), jnp.float32),
        mesh=mesh,
        scratch_types=[
            pltpu.VMEM((b_per_w,), jnp.int32),
            pltpu.VMEM((NBUF, CHUNK, EMBED_DIM), jnp.float32),
            pltpu.SemaphoreType.DMA((NBUF,)),
            pltpu.SemaphoreType.DMA((NBUF,)),
        ],
    )
    return run(ids_flat, scaled_table)


def kernel(token_ids, table):
    b, s = token_ids.shape
    ids_flat = token_ids.reshape(-1).astype(jnp.int32)
    out = _gather(ids_flat, _prescale(table))
    return out.reshape(b, s, EMBED_DIM)
